# baseline (speedup 1.0000x reference)
import functools

import jax
import jax.numpy as jnp
from jax import lax
from jax.experimental import pallas as pl
from jax.experimental.pallas import tpu as pltpu
from jax.experimental.pallas import tpu_sc as plsc

NC = 2
NS = 16
NW = NC * NS
C = 40
R = 80


def _node_split(n):
  assert n % R == 0
  total = n // R
  nk_hi = -(-total // NS)
  n_hi = total - NS * (nk_hi - 1)
  assert n_hi >= 1
  return nk_hi, n_hi


def _make_deg_kernel(n_nodes, n_edges):
  nch = n_edges // (NW * C)
  nk_hi, n_hi = _node_split(n_nodes)
  mesh = plsc.VectorSubcoreMesh(core_axis_name="c", subcore_axis_name="s")

  @functools.partial(
      pl.kernel,
      out_type=jax.ShapeDtypeStruct((NC * n_nodes,), jnp.float32),
      mesh=mesh,
      scratch_types=[
          pltpu.VMEM((nch, C), jnp.int32),
          pltpu.VMEM((128,), jnp.float32),
          pltpu.VMEM((R,), jnp.float32),
          pltpu.VMEM_SHARED((n_nodes,), jnp.float32),
      ],
  )
  def deg_kernel(row_hbm, out_hbm, rowv, onesv, dbuf, acc):
    c = lax.axis_index("c")
    s = lax.axis_index("s")
    wid = s * NC + c
    lo = R * (nk_hi * s - jnp.maximum(s - n_hi, 0))
    nk = lax.select(s < n_hi, nk_hi, nk_hi - 1)

    for j in range(128 // 16):
      onesv[pl.ds(j * 16, 16)] = jnp.full((16,), 1.0, jnp.float32)
    for j in range(R // 16):
      dbuf[pl.ds(j * 16, 16)] = jnp.zeros((16,), jnp.float32)

    def zinit(k, carry):
      pltpu.sync_copy(dbuf, acc.at[pl.ds(pl.multiple_of(lo + k * R, 8), R)])
      return carry

    lax.fori_loop(0, nk, zinit, 0)
    pltpu.sync_copy(row_hbm.at[wid], rowv)
    plsc.subcore_barrier()

    def body(k, carry):
      pltpu.sync_copy(onesv.at[pl.ds(0, C)], acc.at[rowv.at[k]], add=True)
      return carry

    lax.fori_loop(0, nch, body, 0)
    plsc.subcore_barrier()

    def drain(k, carry):
      src = pl.multiple_of(lo + k * R, 8)
      dst = pl.multiple_of(c * n_nodes + lo + k * R, 8)
      pltpu.sync_copy(acc.at[pl.ds(src, R)], dbuf)
      pltpu.sync_copy(dbuf, out_hbm.at[pl.ds(dst, R)])
      return carry

    lax.fori_loop(0, nk, drain, 0)

  return deg_kernel


NBUF = 8
DIST = 7


def _make_scatter_kernel(n_nodes, n_edges, d):
  nch = n_edges // (NW * C)
  nk_hi, n_hi = _node_split(n_nodes)
  niter = (nch + NBUF - 1) // NBUF
  mesh = plsc.VectorSubcoreMesh(core_axis_name="c", subcore_axis_name="s")
  assert nch >= NBUF + DIST

  @functools.partial(
      pl.kernel,
      out_type=jax.ShapeDtypeStruct((NC * n_nodes, d), jnp.float32),
      mesh=mesh,
      scratch_types=(
          [pltpu.VMEM((2, C), jnp.int32)] * NBUF
          + [pltpu.VMEM((C, d), jnp.float32)] * NBUF
          + [pltpu.VMEM_SHARED((n_nodes, d), jnp.float32)]
          + [pltpu.SemaphoreType.DMA] * (2 * NBUF)
      ),
  )
  def scatter_kernel(idx_hbm, xs_hbm, out_hbm, *refs):
    ibufs = refs[0:NBUF]
    gbufs = refs[NBUF:2 * NBUF]
    acc = refs[2 * NBUF]
    gsems = refs[2 * NBUF + 1:3 * NBUF + 1]
    ssems = refs[3 * NBUF + 1:]
    sbuf = gbufs[0].at[pl.ds(0, R)]
    c = lax.axis_index("c")
    s = lax.axis_index("s")
    wid = s * NC + c
    lo = R * (nk_hi * s - jnp.maximum(s - n_hi, 0))
    cbase = wid * nch
    nk = lax.select(s < n_hi, nk_hi, nk_hi - 1)

    def gather_desc(b):
      return pltpu.make_async_copy(xs_hbm.at[ibufs[b].at[1]], gbufs[b],
                                   gsems[b])

    def scatter_desc(b):
      return pltpu.make_async_copy(gbufs[b], acc.at[ibufs[b].at[0]],
                                   ssems[b])

    def init(k, carry):
      base = pl.multiple_of(lo + k * R, 8)
      pltpu.sync_copy(xs_hbm.at[pl.ds(base, R)], sbuf)
      pltpu.sync_copy(sbuf, acc.at[pl.ds(base, R)])
      return carry

    lax.fori_loop(0, nk, init, 0)
    for b in range(DIST):
      pltpu.sync_copy(idx_hbm.at[cbase + b], ibufs[b])
      pltpu.async_copy(xs_hbm.at[ibufs[b].at[1]], gbufs[b], gsems[b])
    plsc.subcore_barrier()

    def body(i, carry):
      for b in range(NBUF):
        k = i * NBUF + b

        @pl.when(k < nch)
        def _():
          gather_desc(b).wait()
          pltpu.async_copy(gbufs[b], acc.at[ibufs[b].at[0]], ssems[b],
                           add=True)
        k2 = k + DIST
        b2 = (b + DIST) % NBUF

        @pl.when(k2 < nch)
        def _():
          @pl.when(k2 >= NBUF)
          def _():
            scatter_desc(b2).wait()
          pltpu.sync_copy(idx_hbm.at[cbase + k2], ibufs[b2])
          pltpu.async_copy(xs_hbm.at[ibufs[b2].at[1]], gbufs[b2], gsems[b2])
      return carry

    lax.fori_loop(0, niter, body, 0)
    for t in range(NBUF):
      scatter_desc((nch - NBUF + t) % NBUF).wait()
    plsc.subcore_barrier()

    def drain(k, carry):
      src = pl.multiple_of(lo + k * R, 8)
      dst = pl.multiple_of(c * n_nodes + lo + k * R, 8)
      pltpu.sync_copy(acc.at[pl.ds(src, R)], sbuf)
      pltpu.sync_copy(sbuf, out_hbm.at[pl.ds(dst, R)])
      return carry

    lax.fori_loop(0, nk, drain, 0)

  return scatter_kernel


def _pre(deg2, x, block):
  n, d = x.shape

  def body(deg_ref, x_ref, xs_ref, dinv_ref):
    dinv = lax.rsqrt(deg_ref[0] + deg_ref[1] + 1.0)
    dinv_b = jnp.broadcast_to(dinv, (block, d))
    xs_ref[...] = x_ref[...] * dinv_b
    dinv_ref[...] = dinv_b

  return pl.pallas_call(
      body,
      grid=(n // block,),
      in_specs=[
          pl.BlockSpec((2, block, 1), lambda i: (0, i, 0)),
          pl.BlockSpec((block, d), lambda i: (i, 0)),
      ],
      out_specs=[
          pl.BlockSpec((block, d), lambda i: (i, 0)),
          pl.BlockSpec((block, d), lambda i: (i, 0)),
      ],
      out_shape=[
          jax.ShapeDtypeStruct((n, d), jnp.float32),
          jax.ShapeDtypeStruct((n, d), jnp.float32),
      ],
  )(deg2, x)


def _post(parts, xs, dinv, wt, block, relu_rescale):
  _, n, d = parts.shape
  dout = wt.shape[1]

  def body(p_ref, xs_ref, d_ref, w_ref, o_ref):
    agg = (p_ref[0] + p_ref[1] - xs_ref[...]) * d_ref[...]
    y = jnp.dot(agg, w_ref[...], preferred_element_type=jnp.float32)
    if relu_rescale:
      y = jnp.maximum(y, 0.0) * d_ref[...]
    o_ref[...] = y

  return pl.pallas_call(
      body,
      grid=(n // block,),
      in_specs=[
          pl.BlockSpec((2, block, d), lambda i: (0, i, 0)),
          pl.BlockSpec((block, d), lambda i: (i, 0)),
          pl.BlockSpec((block, d), lambda i: (i, 0)),
          pl.BlockSpec((d, dout), lambda i: (0, 0)),
      ],
      out_specs=pl.BlockSpec((block, dout), lambda i: (i, 0)),
      out_shape=jax.ShapeDtypeStruct((n, dout), jnp.float32),
  )(parts, xs, dinv, wt)


def kernel(x, edge_index, W_pre, W_out):
  n, d = x.shape
  e = edge_index.shape[1]

  nch = e // (NW * C)
  row3 = edge_index[0].reshape(NW, nch, C)
  col3 = edge_index[1].reshape(NW, nch, C)
  idx3 = jnp.stack([row3, col3], axis=2).reshape(NW * nch, 2, C)

  deg2 = _make_deg_kernel(n, e)(row3)
  xs1, dinv = _pre(deg2.reshape(NC, n, 1), x, 2000)

  scat = _make_scatter_kernel(n, e, d)
  parts1 = scat(idx3, xs1).reshape(NC, n, d)
  xs2 = _post(parts1, xs1, dinv, W_pre.T, 2000, True)
  parts2 = scat(idx3, xs2).reshape(NC, n, d)
  return _post(parts2, xs2, dinv, W_out.T, 2000, False)


# speedup vs baseline: 1.4620x; 1.4620x over previous
import functools

import jax
import jax.numpy as jnp
from jax import lax
from jax.experimental import pallas as pl
from jax.experimental.pallas import tpu as pltpu
from jax.experimental.pallas import tpu_sc as plsc

NC = 2
NS = 16
NW = NC * NS
C = 80
R = 80


def _node_split(n):
  assert n % R == 0
  total = n // R
  nk_hi = -(-total // NS)
  n_hi = total - NS * (nk_hi - 1)
  assert n_hi >= 1
  return nk_hi, n_hi


def _make_deg_kernel(n_nodes, n_edges):
  nch = n_edges // (NW * C)
  nk_hi, n_hi = _node_split(n_nodes)
  mesh = plsc.VectorSubcoreMesh(core_axis_name="c", subcore_axis_name="s")

  @functools.partial(
      pl.kernel,
      out_type=jax.ShapeDtypeStruct((NC * n_nodes,), jnp.float32),
      mesh=mesh,
      scratch_types=[
          pltpu.VMEM((nch, C), jnp.int32),
          pltpu.VMEM((128,), jnp.float32),
          pltpu.VMEM((R,), jnp.float32),
          pltpu.VMEM_SHARED((n_nodes,), jnp.float32),
      ],
  )
  def deg_kernel(row_hbm, out_hbm, rowv, onesv, dbuf, acc):
    c = lax.axis_index("c")
    s = lax.axis_index("s")
    wid = s * NC + c
    lo = R * (nk_hi * s - jnp.maximum(s - n_hi, 0))
    nk = lax.select(s < n_hi, nk_hi, nk_hi - 1)

    for j in range(128 // 16):
      onesv[pl.ds(j * 16, 16)] = jnp.full((16,), 1.0, jnp.float32)
    for j in range(R // 16):
      dbuf[pl.ds(j * 16, 16)] = jnp.zeros((16,), jnp.float32)

    def zinit(k, carry):
      pltpu.sync_copy(dbuf, acc.at[pl.ds(pl.multiple_of(lo + k * R, 8), R)])
      return carry

    lax.fori_loop(0, nk, zinit, 0)
    pltpu.sync_copy(row_hbm.at[wid], rowv)
    plsc.subcore_barrier()

    def body(k, carry):
      pltpu.sync_copy(onesv.at[pl.ds(0, C)], acc.at[rowv.at[k]], add=True)
      return carry

    lax.fori_loop(0, nch, body, 0)
    plsc.subcore_barrier()

    def drain(k, carry):
      src = pl.multiple_of(lo + k * R, 8)
      dst = pl.multiple_of(c * n_nodes + lo + k * R, 8)
      pltpu.sync_copy(acc.at[pl.ds(src, R)], dbuf)
      pltpu.sync_copy(dbuf, out_hbm.at[pl.ds(dst, R)])
      return carry

    lax.fori_loop(0, nk, drain, 0)

  return deg_kernel


NBUF = 4
DIST = 3


def _make_scatter_kernel(n_nodes, n_edges, d):
  nch = n_edges // (NW * C)
  nk_hi, n_hi = _node_split(n_nodes)
  niter = (nch + NBUF - 1) // NBUF
  mesh = plsc.VectorSubcoreMesh(core_axis_name="c", subcore_axis_name="s")
  assert nch >= NBUF + DIST

  @functools.partial(
      pl.kernel,
      out_type=jax.ShapeDtypeStruct((NC * n_nodes, d), jnp.float32),
      mesh=mesh,
      scratch_types=(
          [pltpu.VMEM((2, C), jnp.int32)] * NBUF
          + [pltpu.VMEM((C, d), jnp.float32)] * NBUF
          + [pltpu.VMEM_SHARED((n_nodes, d), jnp.float32)]
          + [pltpu.SemaphoreType.DMA] * (2 * NBUF)
      ),
  )
  def scatter_kernel(idx_hbm, xs_hbm, out_hbm, *refs):
    ibufs = refs[0:NBUF]
    gbufs = refs[NBUF:2 * NBUF]
    acc = refs[2 * NBUF]
    gsems = refs[2 * NBUF + 1:3 * NBUF + 1]
    ssems = refs[3 * NBUF + 1:]
    sbuf = gbufs[0].at[pl.ds(0, R)]
    c = lax.axis_index("c")
    s = lax.axis_index("s")
    wid = s * NC + c
    lo = R * (nk_hi * s - jnp.maximum(s - n_hi, 0))
    cbase = wid * nch
    nk = lax.select(s < n_hi, nk_hi, nk_hi - 1)

    def gather_desc(b):
      return pltpu.make_async_copy(xs_hbm.at[ibufs[b].at[1]], gbufs[b],
                                   gsems[b])

    def scatter_desc(b):
      return pltpu.make_async_copy(gbufs[b], acc.at[ibufs[b].at[0]],
                                   ssems[b])

    def pipe2(nk_s, src_fn, dst_fn):
      def bufat(b):
        return gbufs[b].at[pl.ds(0, R)]

      for k in range(nk_s + 1):
        b = k % NBUF
        bp = (k - 1) % NBUF
        if k < nk_s:
          if k >= NBUF:
            pltpu.make_async_copy(bufat(b), dst_fn(k - NBUF),
                                  ssems[b]).wait()
          pltpu.async_copy(src_fn(k), bufat(b), gsems[b])
        if k >= 1:
          pltpu.make_async_copy(src_fn(k - 1), bufat(bp), gsems[bp]).wait()
          pltpu.async_copy(bufat(bp), dst_fn(k - 1), ssems[bp])
      for t in range(min(NBUF, nk_s)):
        k = nk_s - min(NBUF, nk_s) + t
        pltpu.make_async_copy(bufat(k % NBUF), dst_fn(k),
                              ssems[k % NBUF]).wait()

    def rowslice(ref, off):
      def fn(k):
        return ref.at[pl.ds(pl.multiple_of(off + lo + k * R, 8), R)]
      return fn

    @pl.when(s < n_hi)
    def _():
      pipe2(nk_hi, rowslice(xs_hbm, 0), rowslice(acc, 0))

    @pl.when(s >= n_hi)
    def _():
      pipe2(nk_hi - 1, rowslice(xs_hbm, 0), rowslice(acc, 0))
    for b in range(DIST):
      pltpu.sync_copy(idx_hbm.at[cbase + b], ibufs[b])
      pltpu.async_copy(xs_hbm.at[ibufs[b].at[1]], gbufs[b], gsems[b])
    plsc.subcore_barrier()

    def body(i, carry):
      for b in range(NBUF):
        k = i * NBUF + b

        @pl.when(k < nch)
        def _():
          gather_desc(b).wait()
          pltpu.async_copy(gbufs[b], acc.at[ibufs[b].at[0]], ssems[b],
                           add=True)
        k2 = k + DIST
        b2 = (b + DIST) % NBUF

        @pl.when(k2 < nch)
        def _():
          @pl.when(k2 >= NBUF)
          def _():
            scatter_desc(b2).wait()
          pltpu.sync_copy(idx_hbm.at[cbase + k2], ibufs[b2])
          pltpu.async_copy(xs_hbm.at[ibufs[b2].at[1]], gbufs[b2], gsems[b2])
      return carry

    lax.fori_loop(0, niter, body, 0)
    for t in range(NBUF):
      scatter_desc((nch - NBUF + t) % NBUF).wait()
    plsc.subcore_barrier()

    @pl.when(s < n_hi)
    def _():
      pipe2(nk_hi, rowslice(acc, 0), rowslice(out_hbm, c * n_nodes))

    @pl.when(s >= n_hi)
    def _():
      pipe2(nk_hi - 1, rowslice(acc, 0), rowslice(out_hbm, c * n_nodes))

  return scatter_kernel


def _pre(deg2, x, block):
  n, d = x.shape

  def body(deg_ref, x_ref, xs_ref, dinv_ref):
    dinv = lax.rsqrt(deg_ref[0] + deg_ref[1] + 1.0)
    dinv_b = jnp.broadcast_to(dinv, (block, d))
    xs_ref[...] = x_ref[...] * dinv_b
    dinv_ref[...] = dinv_b

  return pl.pallas_call(
      body,
      grid=(n // block,),
      in_specs=[
          pl.BlockSpec((2, block, 1), lambda i: (0, i, 0)),
          pl.BlockSpec((block, d), lambda i: (i, 0)),
      ],
      out_specs=[
          pl.BlockSpec((block, d), lambda i: (i, 0)),
          pl.BlockSpec((block, d), lambda i: (i, 0)),
      ],
      out_shape=[
          jax.ShapeDtypeStruct((n, d), jnp.float32),
          jax.ShapeDtypeStruct((n, d), jnp.float32),
      ],
  )(deg2, x)


def _post(parts, xs, dinv, wt, block, relu_rescale):
  _, n, d = parts.shape
  dout = wt.shape[1]

  def body(p_ref, xs_ref, d_ref, w_ref, o_ref):
    agg = (p_ref[0] + p_ref[1] - xs_ref[...]) * d_ref[...]
    y = jnp.dot(agg, w_ref[...], preferred_element_type=jnp.float32)
    if relu_rescale:
      y = jnp.maximum(y, 0.0) * d_ref[...]
    o_ref[...] = y

  return pl.pallas_call(
      body,
      grid=(n // block,),
      in_specs=[
          pl.BlockSpec((2, block, d), lambda i: (0, i, 0)),
          pl.BlockSpec((block, d), lambda i: (i, 0)),
          pl.BlockSpec((block, d), lambda i: (i, 0)),
          pl.BlockSpec((d, dout), lambda i: (0, 0)),
      ],
      out_specs=pl.BlockSpec((block, dout), lambda i: (i, 0)),
      out_shape=jax.ShapeDtypeStruct((n, dout), jnp.float32),
  )(parts, xs, dinv, wt)


def kernel(x, edge_index, W_pre, W_out):
  n, d = x.shape
  e = edge_index.shape[1]

  nch = e // (NW * C)
  row3 = edge_index[0].reshape(NW, nch, C)
  col3 = edge_index[1].reshape(NW, nch, C)
  idx3 = jnp.stack([row3, col3], axis=2).reshape(NW * nch, 2, C)

  deg2 = _make_deg_kernel(n, e)(row3)
  xs1, dinv = _pre(deg2.reshape(NC, n, 1), x, 2000)

  scat = _make_scatter_kernel(n, e, d)
  parts1 = scat(idx3, xs1).reshape(NC, n, d)
  xs2 = _post(parts1, xs1, dinv, W_pre.T, 2000, True)
  parts2 = scat(idx3, xs2).reshape(NC, n, d)
  return _post(parts2, xs2, dinv, W_out.T, 2000, False)


# speedup vs baseline: 1.4930x; 1.0212x over previous
import functools

import jax
import jax.numpy as jnp
from jax import lax
from jax.experimental import pallas as pl
from jax.experimental.pallas import tpu as pltpu
from jax.experimental.pallas import tpu_sc as plsc

NC = 2
NS = 16
NW = NC * NS
C = 80
R = 80


def _node_split(n):
  assert n % R == 0
  total = n // R
  nk_hi = -(-total // NS)
  n_hi = total - NS * (nk_hi - 1)
  assert n_hi >= 1
  return nk_hi, n_hi


def _make_deg_kernel(n_nodes, n_edges):
  nch = n_edges // (NW * C)
  nk_hi, n_hi = _node_split(n_nodes)
  mesh = plsc.VectorSubcoreMesh(core_axis_name="c", subcore_axis_name="s")

  @functools.partial(
      pl.kernel,
      out_type=jax.ShapeDtypeStruct((NC * n_nodes,), jnp.float32),
      mesh=mesh,
      scratch_types=[
          pltpu.VMEM((nch, C), jnp.int32),
          pltpu.VMEM((128,), jnp.float32),
          pltpu.VMEM((R,), jnp.float32),
          pltpu.VMEM_SHARED((n_nodes,), jnp.float32),
          pltpu.SemaphoreType.DMA,
      ],
  )
  def deg_kernel(row_hbm, out_hbm, rowv, onesv, dbuf, acc, sem):
    c = lax.axis_index("c")
    s = lax.axis_index("s")
    wid = s * NC + c
    lo = R * (nk_hi * s - jnp.maximum(s - n_hi, 0))
    nk = lax.select(s < n_hi, nk_hi, nk_hi - 1)

    for j in range(128 // 16):
      onesv[pl.ds(j * 16, 16)] = jnp.full((16,), 1.0, jnp.float32)
    for j in range(R // 16):
      dbuf[pl.ds(j * 16, 16)] = jnp.zeros((16,), jnp.float32)

    def zinit(k, carry):
      pltpu.sync_copy(dbuf, acc.at[pl.ds(pl.multiple_of(lo + k * R, 8), R)])
      return carry

    lax.fori_loop(0, nk, zinit, 0)
    pltpu.sync_copy(row_hbm.at[wid], rowv)
    plsc.subcore_barrier()

    def body(k, carry):
      @pl.when(k >= 4)
      def _():
        pltpu.make_async_copy(onesv.at[pl.ds(0, C)], acc.at[rowv.at[k - 4]],
                              sem).wait()
      pltpu.async_copy(onesv.at[pl.ds(0, C)], acc.at[rowv.at[k]], sem,
                       add=True)
      return carry

    lax.fori_loop(0, nch, body, 0)
    for t in range(min(4, nch)):
      pltpu.make_async_copy(onesv.at[pl.ds(0, C)],
                            acc.at[rowv.at[nch - min(4, nch) + t]],
                            sem).wait()
    plsc.subcore_barrier()

    def drain(k, carry):
      src = pl.multiple_of(lo + k * R, 8)
      dst = pl.multiple_of(c * n_nodes + lo + k * R, 8)
      pltpu.sync_copy(acc.at[pl.ds(src, R)], dbuf)
      pltpu.sync_copy(dbuf, out_hbm.at[pl.ds(dst, R)])
      return carry

    lax.fori_loop(0, nk, drain, 0)

  return deg_kernel


NBUF = 4
DIST = 3


def _make_scatter_kernel(n_nodes, n_edges, d):
  nch = n_edges // (NW * C)
  nk_hi, n_hi = _node_split(n_nodes)
  niter = (nch + NBUF - 1) // NBUF
  mesh = plsc.VectorSubcoreMesh(core_axis_name="c", subcore_axis_name="s")
  assert nch >= NBUF + DIST

  @functools.partial(
      pl.kernel,
      out_type=jax.ShapeDtypeStruct((NC * n_nodes, d), jnp.float32),
      mesh=mesh,
      scratch_types=(
          [pltpu.VMEM((2, C), jnp.int32)] * NBUF
          + [pltpu.VMEM((C, d), jnp.float32)] * NBUF
          + [pltpu.VMEM_SHARED((n_nodes, d), jnp.float32)]
          + [pltpu.SemaphoreType.DMA] * (2 * NBUF)
      ),
  )
  def scatter_kernel(idx_hbm, xs_hbm, out_hbm, *refs):
    ibufs = refs[0:NBUF]
    gbufs = refs[NBUF:2 * NBUF]
    acc = refs[2 * NBUF]
    gsems = refs[2 * NBUF + 1:3 * NBUF + 1]
    ssems = refs[3 * NBUF + 1:]
    sbuf = gbufs[0].at[pl.ds(0, R)]
    c = lax.axis_index("c")
    s = lax.axis_index("s")
    wid = s * NC + c
    lo = R * (nk_hi * s - jnp.maximum(s - n_hi, 0))
    cbase = wid * nch
    nk = lax.select(s < n_hi, nk_hi, nk_hi - 1)

    def gather_desc(b):
      return pltpu.make_async_copy(xs_hbm.at[ibufs[b].at[1]], gbufs[b],
                                   gsems[b])

    def scatter_desc(b):
      return pltpu.make_async_copy(gbufs[b], acc.at[ibufs[b].at[0]],
                                   ssems[b])

    def pipe2(nk_s, src_fn, dst_fn):
      def bufat(b):
        return gbufs[b].at[pl.ds(0, R)]

      for k in range(nk_s + 1):
        b = k % NBUF
        bp = (k - 1) % NBUF
        if k < nk_s:
          if k >= NBUF:
            pltpu.make_async_copy(bufat(b), dst_fn(k - NBUF),
                                  ssems[b]).wait()
          pltpu.async_copy(src_fn(k), bufat(b), gsems[b])
        if k >= 1:
          pltpu.make_async_copy(src_fn(k - 1), bufat(bp), gsems[bp]).wait()
          pltpu.async_copy(bufat(bp), dst_fn(k - 1), ssems[bp])
      for t in range(min(NBUF, nk_s)):
        k = nk_s - min(NBUF, nk_s) + t
        pltpu.make_async_copy(bufat(k % NBUF), dst_fn(k),
                              ssems[k % NBUF]).wait()

    def rowslice(ref, off):
      def fn(k):
        return ref.at[pl.ds(pl.multiple_of(off + lo + k * R, 8), R)]
      return fn

    @pl.when(s < n_hi)
    def _():
      pipe2(nk_hi, rowslice(xs_hbm, 0), rowslice(acc, 0))

    @pl.when(s >= n_hi)
    def _():
      pipe2(nk_hi - 1, rowslice(xs_hbm, 0), rowslice(acc, 0))
    for b in range(DIST):
      pltpu.sync_copy(idx_hbm.at[cbase + b], ibufs[b])
      pltpu.async_copy(xs_hbm.at[ibufs[b].at[1]], gbufs[b], gsems[b])
    plsc.subcore_barrier()

    def body(i, carry):
      for b in range(NBUF):
        k = i * NBUF + b

        @pl.when(k < nch)
        def _():
          gather_desc(b).wait()
          pltpu.async_copy(gbufs[b], acc.at[ibufs[b].at[0]], ssems[b],
                           add=True)
        k2 = k + DIST
        b2 = (b + DIST) % NBUF

        @pl.when(k2 < nch)
        def _():
          @pl.when(k2 >= NBUF)
          def _():
            scatter_desc(b2).wait()
          pltpu.sync_copy(idx_hbm.at[cbase + k2], ibufs[b2])
          pltpu.async_copy(xs_hbm.at[ibufs[b2].at[1]], gbufs[b2], gsems[b2])
      return carry

    lax.fori_loop(0, niter, body, 0)
    for t in range(NBUF):
      scatter_desc((nch - NBUF + t) % NBUF).wait()
    plsc.subcore_barrier()

    @pl.when(s < n_hi)
    def _():
      pipe2(nk_hi, rowslice(acc, 0), rowslice(out_hbm, c * n_nodes))

    @pl.when(s >= n_hi)
    def _():
      pipe2(nk_hi - 1, rowslice(acc, 0), rowslice(out_hbm, c * n_nodes))

  return scatter_kernel


def _pre(deg2, x, block):
  n, d = x.shape

  def body(deg_ref, x_ref, xs_ref, dinv_ref):
    dinv = lax.rsqrt(deg_ref[0] + deg_ref[1] + 1.0)
    dinv_b = jnp.broadcast_to(dinv, (block, d))
    xs_ref[...] = x_ref[...] * dinv_b
    dinv_ref[...] = dinv_b

  return pl.pallas_call(
      body,
      grid=(n // block,),
      in_specs=[
          pl.BlockSpec((2, block, 1), lambda i: (0, i, 0)),
          pl.BlockSpec((block, d), lambda i: (i, 0)),
      ],
      out_specs=[
          pl.BlockSpec((block, d), lambda i: (i, 0)),
          pl.BlockSpec((block, d), lambda i: (i, 0)),
      ],
      out_shape=[
          jax.ShapeDtypeStruct((n, d), jnp.float32),
          jax.ShapeDtypeStruct((n, d), jnp.float32),
      ],
  )(deg2, x)


def _post(parts, xs, dinv, wt, block, relu_rescale):
  _, n, d = parts.shape
  dout = wt.shape[1]

  def body(p_ref, xs_ref, d_ref, w_ref, o_ref):
    agg = (p_ref[0] + p_ref[1] - xs_ref[...]) * d_ref[...]
    y = jnp.dot(agg, w_ref[...], preferred_element_type=jnp.float32)
    if relu_rescale:
      y = jnp.maximum(y, 0.0) * d_ref[...]
    o_ref[...] = y

  return pl.pallas_call(
      body,
      grid=(n // block,),
      in_specs=[
          pl.BlockSpec((2, block, d), lambda i: (0, i, 0)),
          pl.BlockSpec((block, d), lambda i: (i, 0)),
          pl.BlockSpec((block, d), lambda i: (i, 0)),
          pl.BlockSpec((d, dout), lambda i: (0, 0)),
      ],
      out_specs=pl.BlockSpec((block, dout), lambda i: (i, 0)),
      out_shape=jax.ShapeDtypeStruct((n, dout), jnp.float32),
  )(parts, xs, dinv, wt)


def kernel(x, edge_index, W_pre, W_out):
  n, d = x.shape
  e = edge_index.shape[1]

  nch = e // (NW * C)
  row3 = edge_index[0].reshape(NW, nch, C)
  col3 = edge_index[1].reshape(NW, nch, C)
  idx3 = jnp.stack([row3, col3], axis=2).reshape(NW * nch, 2, C)

  deg2 = _make_deg_kernel(n, e)(row3)
  xs1, dinv = _pre(deg2.reshape(NC, n, 1), x, 2000)

  scat = _make_scatter_kernel(n, e, d)
  parts1 = scat(idx3, xs1).reshape(NC, n, d)
  xs2 = _post(parts1, xs1, dinv, W_pre.T, 2000, True)
  parts2 = scat(idx3, xs2).reshape(NC, n, d)
  return _post(parts2, xs2, dinv, W_out.T, 2000, False)
